# baseline (speedup 1.0000x reference)
import functools
import math

import jax
import jax.numpy as jnp
from jax import lax
from jax.experimental import pallas as pl
from jax.experimental.pallas import tpu as pltpu, tpu_sc as plsc

_B = 16
_T = 8
_NC = 30
_N = 2100
_CH = 35
_ROWSZ = _CH * _N
_NCHUNK = 60
_GRIDS = ((40, 0), (20, 1600), (10, 2000))

_DN = lax.GatherDimensionNumbers(offset_dims=(), collapsed_slice_dims=(0,),
                                 start_index_map=(0,))


def _lane():
    return lax.iota(jnp.int32, 16)


def _shuf(x, idx):
    return lax.gather(x, idx[:, None], _DN, (1,),
                      mode=lax.GatherScatterMode.PROMISE_IN_BOUNDS)


def _btotal(x):
    lane = _lane()
    for m in (1, 2, 4, 8):
        x = x + _shuf(x, jnp.bitwise_xor(lane, m))
    return x


_LOG1P_C = (0.9999998140790242, -0.49999158439244973, 0.3331932856366838,
            -0.24879664337291735, 0.193819406790088, -0.14607315633452525,
            0.09585679177503667, -0.04804633240211843, 0.015551125988479492,
            -0.002365529162657254)


def _sp(x):
    e = jnp.exp(-jnp.abs(x))
    p = jnp.float32(_LOG1P_C[-1])
    for c in _LOG1P_C[-2::-1]:
        p = p * e + jnp.float32(c)
    return jnp.maximum(x, 0.0) + p * e


_ATAN_C = (0.9999994160035323, -0.3333022235532034, 0.19951110891900398,
           -0.139332293932798, 0.0970935073714827, -0.05688089274199308,
           0.022566826126643333, -0.004257409078054553)


def _atan(x):
    t = jnp.abs(x)
    inv = t > 1.0
    z = jnp.where(inv, 1.0 / jnp.maximum(t, 1e-30), t)
    u = z * z
    p = jnp.float32(_ATAN_C[-1])
    for c in _ATAN_C[-2::-1]:
        p = p * u + jnp.float32(c)
    r = z * p
    r = jnp.where(inv, jnp.float32(math.pi / 2) - r, r)
    return jnp.sign(x) * r


def _extract_col(t0, t1, t2, col):
    j = jnp.minimum(_lane(), 7)
    fp = j * 5 + col
    g0 = _shuf(t0, jnp.clip(fp, 0, 15))
    g1 = _shuf(t1, jnp.clip(fp - 16, 0, 15))
    g2 = _shuf(t2, jnp.clip(fp - 24, 0, 15))
    return jnp.where(fp < 16, g0, jnp.where(fp < 32, g1, g2))


def _select_top3(px, py):
    f32 = jnp.float32
    lane = _lane()
    inf = lane.astype(f32) * 0.0 + jnp.float32(1e30)
    big = lane * 0 + (1 << 30)
    d1, d2, d3 = inf, inf, inf
    n1, n2, n3 = big, big, big
    for g, base in _GRIDS:
        gf = f32(g)
        cx = (px * gf).astype(jnp.int32)
        cy = (py * gf).astype(jnp.int32)
        for dy in (-1, 0, 1):
            for dx in (-1, 0, 1):
                xx = jnp.clip(cx + dx, 0, g - 1)
                yy = jnp.clip(cy + dy, 0, g - 1)
                n = base + yy * g + xx
                ctrx = (xx.astype(f32) + 0.5) / gf
                ctry = (yy.astype(f32) + 0.5) / gf
                ddx = px - ctrx
                ddy = py - ctry
                d = ddx * ddx + ddy * ddy
                def _w(b):
                    return jnp.where(b, 1, 0)
                dup = jnp.maximum(jnp.maximum(_w(n == n1), _w(n == n2)),
                                  _w(n == n3))
                def _lt(dk, nk):
                    return jnp.maximum(_w(d < dk), _w(d == dk) * _w(n < nk))
                lt1, lt2, lt3 = _lt(d1, n1), _lt(d2, n2), _lt(d3, n3)
                nd = 1 - dup
                c1 = lt1 * nd
                c2 = (1 - lt1) * lt2 * nd
                c3 = (1 - lt1) * (1 - lt2) * lt3 * nd
                c12 = c1 + c2
                keep3 = 1 - c12 - c3
                c12f = c12.astype(jnp.float32)
                c3f = c3.astype(jnp.float32)
                k3f = keep3.astype(jnp.float32)
                d3 = c12f * d2 + c3f * d + k3f * d3
                n3 = c12 * n2 + c3 * n + keep3 * n3
                c1f = c1.astype(jnp.float32)
                c2f = c2.astype(jnp.float32)
                k2f = (1 - c1 - c2).astype(jnp.float32)
                d2 = c1f * d1 + c2f * d + k2f * d2
                n2 = c1 * n1 + c2 * n + (1 - c1 - c2) * n2
                c1ff = c1f
                d1 = c1ff * d + (1.0 - c1ff) * d1
                n1 = c1 * n + (1 - c1) * n1
    return n1, n2, n3


def _dedup(ns, cls_i):
    lane = _lane()
    keys = [n * _NC + cls_i for n in ns]
    uk, ck = [], []
    for s in range(3):
        du = jnp.zeros((16,), jnp.bool_)
        dc = jnp.zeros((16,), jnp.bool_)
        for s2 in range(s + 1):
            within = s2 == s
            for r in range(8):
                if within and r == 0:
                    continue
                idx = jnp.bitwise_and(lane - r, 7)
                rn = _shuf(ns[s2], idx)
                rk = _shuf(keys[s2], idx)
                if within:
                    valid = lane >= r
                    du = du | ((ns[s] == rn) & valid)
                    dc = dc | ((keys[s] == rk) & valid)
                else:
                    du = du | (ns[s] == rn)
                    dc = dc | (keys[s] == rk)
        uk.append(jnp.where(du, 0.0, 1.0))
        ck.append(jnp.where(dc, 0.0, 1.0))
    return uk, ck


def _ciou_terms(b1, b2):
    b1x, b1y, b1w, b1h = b1
    b2x, b2y, b2w, b2h = b2
    b1x1, b1x2 = b1x - b1w * 0.5, b1x + b1w * 0.5
    b1y1, b1y2 = b1y - b1h * 0.5, b1y + b1h * 0.5
    b2x1, b2x2 = b2x - b2w * 0.5, b2x + b2w * 0.5
    b2y1, b2y2 = b2y - b2h * 0.5, b2y + b2h * 0.5
    iw = jnp.maximum(jnp.minimum(b1x2, b2x2) - jnp.maximum(b1x1, b2x1), 0.0)
    ih = jnp.maximum(jnp.minimum(b1y2, b2y2) - jnp.maximum(b1y1, b2y1), 0.0)
    inter = iw * ih
    union = b1w * b1h + b2w * b2h - inter + 1e-07
    iou = inter / union
    cw = jnp.maximum(b1x2, b2x2) - jnp.minimum(b1x1, b2x1)
    ch = jnp.maximum(b1y2, b2y2) - jnp.minimum(b1y1, b2y1)
    c2 = cw * cw + ch * ch + 1e-07
    rx = b1x1 + b1x2 - b2x1 - b2x2
    ry = b1y1 + b1y2 - b2y1 - b2y2
    rho2 = (rx * rx + ry * ry) * 0.25
    da = _atan(b1w / (b1h + 1e-07)) - _atan(b2w / (b2h + 1e-07))
    v = (4.0 / math.pi ** 2) * da * da
    alpha = v / (1.0 - iou + v + 1e-07)
    return 1.0 - (iou - (rho2 / c2 + v * alpha))


def _sc_loss_call(pred_flat, tgt_flat):
    mesh = plsc.VectorSubcoreMesh(core_axis_name="c", subcore_axis_name="s")

    @functools.partial(
        pl.kernel, mesh=mesh,
        out_type=jax.ShapeDtypeStruct((32, 16), jnp.float32),
        scratch_types=[pltpu.VMEM((40,), jnp.float32),
                       pltpu.VMEM((8, 128), jnp.int32),
                       pltpu.VMEM((8, 128), jnp.float32),
                       pltpu.VMEM((1056,), jnp.float32),
                       pltpu.VMEM((16,), jnp.float32),
                       pltpu.VMEM((16,), jnp.int32),
                       pltpu.SemaphoreType.DMA],
    )
    def k(pred_hbm, tgt_hbm, out_hbm, tgt_v, idx_v, vals_v, row_v, stage_v,
          iscr_v, sem):
        f32 = jnp.float32
        cid = lax.axis_index("c")
        sid = lax.axis_index("s")
        lane = _lane()
        m8f = jnp.where(lane < 8, f32(1.0), f32(0.0))

        pltpu.sync_copy(tgt_hbm.at[pl.ds(sid * 40, 40)], tgt_v)
        t0 = tgt_v[pl.ds(0, 16)]
        t1 = tgt_v[pl.ds(16, 16)]
        t2 = tgt_v[pl.ds(24, 16)]
        cls_f = _extract_col(t0, t1, t2, 0)
        px = _extract_col(t0, t1, t2, 1)
        py = _extract_col(t0, t1, t2, 2)
        tw = _extract_col(t0, t1, t2, 3)
        th = _extract_col(t0, t1, t2, 4)
        cls_i = cls_f.astype(jnp.int32)

        ns = _select_top3(px, py)

        ibase = sid * _ROWSZ
        for c in range(15):
            ch, s = c // 3, c % 3
            idx_v[c // 8, pl.ds((c % 8) * 16, 16)] = ibase + ch * _N + ns[s]
        clsch0 = 5 + cid * 15
        for c in range(15, _NCHUNK):
            l, s = (c - 15) // 3, (c - 15) % 3
            idx_v[c // 8, pl.ds((c % 8) * 16, 16)] = (
                ibase + (clsch0 + l) * _N + ns[s])
        for s in range(3):
            c = 60 + s
            idx_v[c // 8, pl.ds((c % 8) * 16, 16)] = (
                ibase + (5 + cls_i) * _N + ns[s])
        idx_v[7, pl.ds(7 * 16, 16)] = lane * 0
        descs = [pltpu.async_copy(pred_hbm.at[idx_v.at[r]], vals_v.at[r], sem)
                 for r in range(8)]

        acc_sp = jnp.zeros((16,), f32)
        base = sid * _ROWSZ + 4 * _N + cid * 1050
        m = base % 8
        start = pl.multiple_of(base - m, 8)
        pltpu.sync_copy(pred_hbm.at[pl.ds(start, 1056)], row_v)
        iscr_v[...] = lane * 0 + m
        mv = iscr_v[...]
        for c in range(66):
            li = lane + c * 16
            msk = (li >= mv) & (li < mv + 1050)
            acc_sp = acc_sp + jnp.where(msk, _sp(row_v[pl.ds(c * 16, 16)]), 0.0)

        uk, ck = _dedup(ns, cls_i)

        for d in descs:
            d.wait()

        def chunk(c):
            return vals_v[c // 8, pl.ds((c % 8) * 16, 16)]

        iou_acc = jnp.zeros((16,), f32)
        for s in range(3):
            b1 = (chunk(s), chunk(3 + s), chunk(6 + s), chunk(9 + s))
            iou_acc = iou_acc + _ciou_terms(b1, (px, py, tw, th)) * m8f

        corr_acc = jnp.zeros((16,), f32)
        mcnt_acc = jnp.zeros((16,), f32)
        for s in range(3):
            po = chunk(12 + s)
            corr_acc = corr_acc + uk[s] * m8f * (4.0 * _sp(po) - 5.0 * po)
            mcnt_acc = mcnt_acc + uk[s] * m8f

        cls_acc = jnp.zeros((16,), f32)
        for l in range(15):
            for s in range(3):
                p = chunk(15 + l * 3 + s)
                cls_acc = cls_acc + uk[s] * m8f * (_sp(p) - 0.05 * p)
        win_acc = jnp.zeros((16,), f32)
        for s in range(3):
            pw = chunk(60 + s)
            win_acc = win_acc - ck[s] * m8f * (0.9 * pw)


        res = jnp.where(lane == 0, _btotal(iou_acc), 0.0)
        res = res + jnp.where(lane == 1, _btotal(acc_sp), 0.0)
        res = res + jnp.where(lane == 2, _btotal(corr_acc), 0.0)
        res = res + jnp.where(lane == 3, _btotal(cls_acc), 0.0)
        res = res + jnp.where(lane == 4, _btotal(mcnt_acc), 0.0)
        res = res + jnp.where(lane == 5, _btotal(win_acc), 0.0)
        stage_v[...] = res
        pltpu.sync_copy(stage_v, out_hbm.at[sid * 2 + cid])

    return k(pred_flat, tgt_flat)


@jax.jit
def kernel(pred, targets, grids):
    del grids
    out = _sc_loss_call(jax.lax.collapse(pred, 0, 3), targets.reshape(-1))
    t = jnp.sum(out, axis=0)
    iou_sum = t[0] * 0.5
    corr = t[2] * 0.5
    m_cnt = t[4] * 0.5
    sp_sum = t[1]
    cls_sum = t[3] + t[5] * 0.5
    loss_obj = (sp_sum + corr) / jnp.float32(_B * _N)
    loss_cls = cls_sum / (m_cnt * _NC + 1e-12)
    return 10.0 * iou_sum / jnp.float32(_B * _T * 3) + loss_obj + loss_cls


# speedup vs baseline: 1.0246x; 1.0246x over previous
import functools
import math

import jax
import jax.numpy as jnp
from jax import lax
from jax.experimental import pallas as pl
from jax.experimental.pallas import tpu as pltpu, tpu_sc as plsc

_B = 16
_T = 8
_NC = 30
_N = 2100
_CH = 35
_ROWSZ = _CH * _N
_NCHUNK = 60
_GRIDS = ((40, 0), (20, 1600), (10, 2000))

_DN = lax.GatherDimensionNumbers(offset_dims=(), collapsed_slice_dims=(0,),
                                 start_index_map=(0,))


def _lane():
    return lax.iota(jnp.int32, 16)


def _shuf(x, idx):
    return lax.gather(x, idx[:, None], _DN, (1,),
                      mode=lax.GatherScatterMode.PROMISE_IN_BOUNDS)


def _btotal(x):
    lane = _lane()
    for m in (1, 2, 4, 8):
        x = x + _shuf(x, jnp.bitwise_xor(lane, m))
    return x


_LOG1P_C = (0.9999998140790242, -0.49999158439244973, 0.3331932856366838,
            -0.24879664337291735, 0.193819406790088, -0.14607315633452525,
            0.09585679177503667, -0.04804633240211843, 0.015551125988479492,
            -0.002365529162657254)


def _sp(x):
    e = jnp.exp(-jnp.abs(x))
    p = jnp.float32(_LOG1P_C[-1])
    for c in _LOG1P_C[-2::-1]:
        p = p * e + jnp.float32(c)
    return jnp.maximum(x, 0.0) + p * e


_ATAN_C = (0.9999994160035323, -0.3333022235532034, 0.19951110891900398,
           -0.139332293932798, 0.0970935073714827, -0.05688089274199308,
           0.022566826126643333, -0.004257409078054553)


def _atan(x):
    t = jnp.abs(x)
    inv = t > 1.0
    z = jnp.where(inv, 1.0 / jnp.maximum(t, 1e-30), t)
    u = z * z
    p = jnp.float32(_ATAN_C[-1])
    for c in _ATAN_C[-2::-1]:
        p = p * u + jnp.float32(c)
    r = z * p
    r = jnp.where(inv, jnp.float32(math.pi / 2) - r, r)
    return jnp.sign(x) * r


def _extract_col(t0, t1, t2, col):
    j = jnp.minimum(_lane(), 7)
    fp = j * 5 + col
    g0 = _shuf(t0, jnp.clip(fp, 0, 15))
    g1 = _shuf(t1, jnp.clip(fp - 16, 0, 15))
    g2 = _shuf(t2, jnp.clip(fp - 24, 0, 15))
    return jnp.where(fp < 16, g0, jnp.where(fp < 32, g1, g2))


def _select_top3(px, py):
    f32 = jnp.float32
    lane = _lane()
    inf = lane.astype(f32) * 0.0 + jnp.float32(1e30)
    big = lane * 0 + (1 << 30)
    d1, d2, d3 = inf, inf, inf
    n1, n2, n3 = big, big, big
    for g, base in _GRIDS:
        gf = f32(g)
        cx = (px * gf).astype(jnp.int32)
        cy = (py * gf).astype(jnp.int32)
        for dy in (-1, 0, 1):
            for dx in (-1, 0, 1):
                xx = jnp.clip(cx + dx, 0, g - 1)
                yy = jnp.clip(cy + dy, 0, g - 1)
                n = base + yy * g + xx
                ctrx = (xx.astype(f32) + 0.5) / gf
                ctry = (yy.astype(f32) + 0.5) / gf
                ddx = px - ctrx
                ddy = py - ctry
                d = ddx * ddx + ddy * ddy
                def _w(b):
                    return jnp.where(b, 1, 0)
                dup = jnp.maximum(jnp.maximum(_w(n == n1), _w(n == n2)),
                                  _w(n == n3))
                def _lt(dk, nk):
                    return jnp.maximum(_w(d < dk), _w(d == dk) * _w(n < nk))
                lt1, lt2, lt3 = _lt(d1, n1), _lt(d2, n2), _lt(d3, n3)
                nd = 1 - dup
                c1 = lt1 * nd
                c2 = (1 - lt1) * lt2 * nd
                c3 = (1 - lt1) * (1 - lt2) * lt3 * nd
                c12 = c1 + c2
                keep3 = 1 - c12 - c3
                c12f = c12.astype(jnp.float32)
                c3f = c3.astype(jnp.float32)
                k3f = keep3.astype(jnp.float32)
                d3 = c12f * d2 + c3f * d + k3f * d3
                n3 = c12 * n2 + c3 * n + keep3 * n3
                c1f = c1.astype(jnp.float32)
                c2f = c2.astype(jnp.float32)
                k2f = (1 - c1 - c2).astype(jnp.float32)
                d2 = c1f * d1 + c2f * d + k2f * d2
                n2 = c1 * n1 + c2 * n + (1 - c1 - c2) * n2
                c1ff = c1f
                d1 = c1ff * d + (1.0 - c1ff) * d1
                n1 = c1 * n + (1 - c1) * n1
    return n1, n2, n3


def _dedup(ns, cls_i):
    lane = _lane()
    keys = [n * _NC + cls_i for n in ns]
    uk, ck = [], []
    for s in range(3):
        du = jnp.zeros((16,), jnp.bool_)
        dc = jnp.zeros((16,), jnp.bool_)
        for s2 in range(s + 1):
            within = s2 == s
            for r in range(8):
                if within and r == 0:
                    continue
                idx = jnp.bitwise_and(lane - r, 7)
                rn = _shuf(ns[s2], idx)
                rk = _shuf(keys[s2], idx)
                if within:
                    valid = lane >= r
                    du = du | ((ns[s] == rn) & valid)
                    dc = dc | ((keys[s] == rk) & valid)
                else:
                    du = du | (ns[s] == rn)
                    dc = dc | (keys[s] == rk)
        uk.append(jnp.where(du, 0.0, 1.0))
        ck.append(jnp.where(dc, 0.0, 1.0))
    return uk, ck


def _ciou_terms(b1, b2):
    b1x, b1y, b1w, b1h = b1
    b2x, b2y, b2w, b2h = b2
    b1x1, b1x2 = b1x - b1w * 0.5, b1x + b1w * 0.5
    b1y1, b1y2 = b1y - b1h * 0.5, b1y + b1h * 0.5
    b2x1, b2x2 = b2x - b2w * 0.5, b2x + b2w * 0.5
    b2y1, b2y2 = b2y - b2h * 0.5, b2y + b2h * 0.5
    iw = jnp.maximum(jnp.minimum(b1x2, b2x2) - jnp.maximum(b1x1, b2x1), 0.0)
    ih = jnp.maximum(jnp.minimum(b1y2, b2y2) - jnp.maximum(b1y1, b2y1), 0.0)
    inter = iw * ih
    union = b1w * b1h + b2w * b2h - inter + 1e-07
    iou = inter / union
    cw = jnp.maximum(b1x2, b2x2) - jnp.minimum(b1x1, b2x1)
    ch = jnp.maximum(b1y2, b2y2) - jnp.minimum(b1y1, b2y1)
    c2 = cw * cw + ch * ch + 1e-07
    rx = b1x1 + b1x2 - b2x1 - b2x2
    ry = b1y1 + b1y2 - b2y1 - b2y2
    rho2 = (rx * rx + ry * ry) * 0.25
    da = _atan(b1w / (b1h + 1e-07)) - _atan(b2w / (b2h + 1e-07))
    v = (4.0 / math.pi ** 2) * da * da
    alpha = v / (1.0 - iou + v + 1e-07)
    return 1.0 - (iou - (rho2 / c2 + v * alpha))


def _sc_loss_call(pred_flat, tgt_flat):
    mesh = plsc.VectorSubcoreMesh(core_axis_name="c", subcore_axis_name="s")

    @functools.partial(
        pl.kernel, mesh=mesh,
        out_type=jax.ShapeDtypeStruct((32, 16), jnp.float32),
        scratch_types=[pltpu.VMEM((40,), jnp.float32),
                       pltpu.VMEM((8, 128), jnp.int32),
                       pltpu.VMEM((8, 128), jnp.float32),
                       pltpu.VMEM((16,), jnp.float32),
                       pltpu.SemaphoreType.DMA],
    )
    def k(pred_hbm, tgt_hbm, out_hbm, tgt_v, idx_v, vals_v, stage_v, sem):
        f32 = jnp.float32
        cid = lax.axis_index("c")
        sid = lax.axis_index("s")
        lane = _lane()
        m8f = jnp.where(lane < 8, f32(1.0), f32(0.0))

        pltpu.sync_copy(tgt_hbm.at[pl.ds(sid * 40, 40)], tgt_v)
        t0 = tgt_v[pl.ds(0, 16)]
        t1 = tgt_v[pl.ds(16, 16)]
        t2 = tgt_v[pl.ds(24, 16)]
        cls_f = _extract_col(t0, t1, t2, 0)
        px = _extract_col(t0, t1, t2, 1)
        py = _extract_col(t0, t1, t2, 2)
        tw = _extract_col(t0, t1, t2, 3)
        th = _extract_col(t0, t1, t2, 4)
        cls_i = cls_f.astype(jnp.int32)

        ns = _select_top3(px, py)

        ibase = sid * _ROWSZ
        for c in range(15):
            ch, s = c // 3, c % 3
            idx_v[c // 8, pl.ds((c % 8) * 16, 16)] = ibase + ch * _N + ns[s]
        clsch0 = 5 + cid * 15
        for c in range(15, _NCHUNK):
            l, s = (c - 15) // 3, (c - 15) % 3
            idx_v[c // 8, pl.ds((c % 8) * 16, 16)] = (
                ibase + (clsch0 + l) * _N + ns[s])
        for s in range(3):
            c = 60 + s
            idx_v[c // 8, pl.ds((c % 8) * 16, 16)] = (
                ibase + (5 + cls_i) * _N + ns[s])
        idx_v[7, pl.ds(7 * 16, 16)] = lane * 0
        descs = [pltpu.async_copy(pred_hbm.at[idx_v.at[r]], vals_v.at[r], sem)
                 for r in range(8)]

        uk, ck = _dedup(ns, cls_i)

        for d in descs:
            d.wait()

        def chunk(c):
            return vals_v[c // 8, pl.ds((c % 8) * 16, 16)]

        iou_acc = jnp.zeros((16,), f32)
        for s in range(3):
            b1 = (chunk(s), chunk(3 + s), chunk(6 + s), chunk(9 + s))
            iou_acc = iou_acc + _ciou_terms(b1, (px, py, tw, th)) * m8f

        corr_acc = jnp.zeros((16,), f32)
        mcnt_acc = jnp.zeros((16,), f32)
        for s in range(3):
            po = chunk(12 + s)
            corr_acc = corr_acc + uk[s] * m8f * (4.0 * _sp(po) - 5.0 * po)
            mcnt_acc = mcnt_acc + uk[s] * m8f

        cls_acc = jnp.zeros((16,), f32)
        for l in range(15):
            for s in range(3):
                p = chunk(15 + l * 3 + s)
                cls_acc = cls_acc + uk[s] * m8f * (_sp(p) - 0.05 * p)
        win_acc = jnp.zeros((16,), f32)
        for s in range(3):
            pw = chunk(60 + s)
            win_acc = win_acc - ck[s] * m8f * (0.9 * pw)


        res = jnp.where(lane == 0, _btotal(iou_acc), 0.0)
        res = res + jnp.where(lane == 2, _btotal(corr_acc), 0.0)
        res = res + jnp.where(lane == 3, _btotal(cls_acc), 0.0)
        res = res + jnp.where(lane == 4, _btotal(mcnt_acc), 0.0)
        res = res + jnp.where(lane == 5, _btotal(win_acc), 0.0)
        stage_v[...] = res
        pltpu.sync_copy(stage_v, out_hbm.at[sid * 2 + cid])

    return k(pred_flat, tgt_flat)


def _obj_sum_kernel(pobj_ref, out_ref):
    out_ref[0, 0] = jnp.sum(_sp(pobj_ref[...]))


def _obj_sum_call(pobj):
    return pl.pallas_call(
        _obj_sum_kernel,
        out_shape=jax.ShapeDtypeStruct((1, 1), jnp.float32),
        out_specs=pl.BlockSpec(memory_space=pltpu.SMEM),
    )(pobj)


@jax.jit
def kernel(pred, targets, grids):
    del grids
    out = _sc_loss_call(jax.lax.collapse(pred, 0, 3), targets.reshape(-1))
    sp_sum = _obj_sum_call(pred[:, 4, :])[0, 0]
    t = jnp.sum(out, axis=0)
    iou_sum = t[0] * 0.5
    corr = t[2] * 0.5
    m_cnt = t[4] * 0.5
    cls_sum = t[3] + t[5] * 0.5
    loss_obj = (sp_sum + corr) / jnp.float32(_B * _N)
    loss_cls = cls_sum / (m_cnt * _NC + 1e-12)
    return 10.0 * iou_sum / jnp.float32(_B * _T * 3) + loss_obj + loss_cls
